# baseline (speedup 1.0000x reference)
import jax
import jax.numpy as jnp
from jax.experimental import pallas as pl
from jax.experimental.pallas import tpu as pltpu

B, I, H, U = 4096, 128, 512, 512
TB = 1024


def _bf(a):
    return a.astype(jnp.bfloat16)


def _cfc_kernel(inp_ref, hx_ref, ts_ref, wb_ref, bb_ref,
                w1_ref, b1_ref, w2_ref, b2_ref,
                wa_ref, ba_ref, wtb_ref, btb_ref, out_ref, out2_ref):
    wb = _bf(wb_ref[...])
    w1 = _bf(w1_ref[...])
    w2 = _bf(w2_ref[...])
    wa = _bf(wa_ref[...])
    wtb = _bf(wtb_ref[...])
    half = TB // 2
    for k in range(2):
        rows = pl.ds(k * half, half)
        x = jnp.tanh(
            jnp.dot(_bf(inp_ref[rows, :]), wb[:I],
                    preferred_element_type=jnp.float32)
            + jnp.dot(_bf(hx_ref[rows, :]), wb[I:],
                      preferred_element_type=jnp.float32)
            + bb_ref[...]
        )
        xb = _bf(x)
        ff1 = jnp.tanh(jnp.dot(xb, w1,
                               preferred_element_type=jnp.float32) + b1_ref[...])
        ff2 = jnp.tanh(jnp.dot(xb, w2,
                               preferred_element_type=jnp.float32) + b2_ref[...])
        t_a = jnp.dot(xb, wa,
                      preferred_element_type=jnp.float32) + ba_ref[...]
        t_b = jnp.dot(xb, wtb,
                      preferred_element_type=jnp.float32) + btb_ref[...]
        s = jax.nn.sigmoid(t_a * ts_ref[rows, :] + t_b)
        res = ff1 + s * (ff2 - ff1)
        out_ref[rows, :] = res
        out2_ref[rows, :] = res


def kernel(input, hx, ts, Wb, bb, W_ff1, b_ff1, W_ff2, b_ff2, W_ta, b_ta, W_tb, b_tb):
    ts2 = ts[:, None]
    bb2 = bb[None, :]
    b1 = b_ff1[None, :]
    b2 = b_ff2[None, :]
    ba = b_ta[None, :]
    btb = b_tb[None, :]

    whole = lambda shape: pl.BlockSpec(shape, lambda i: (0,) * len(shape))
    out = pl.pallas_call(
        _cfc_kernel,
        grid=(B // TB,),
        in_specs=[
            pl.BlockSpec((TB, I), lambda i: (i, 0)),
            pl.BlockSpec((TB, H), lambda i: (i, 0)),
            pl.BlockSpec((TB, 1), lambda i: (i, 0)),
            whole((I + H, U)),
            whole((1, U)),
            whole((U, H)), whole((1, H)),
            whole((U, H)), whole((1, H)),
            whole((U, H)), whole((1, H)),
            whole((U, H)), whole((1, H)),
        ],
        out_specs=[pl.BlockSpec((TB, H), lambda i: (i, 0)),
                   pl.BlockSpec((TB, H), lambda i: (i, 0))],
        out_shape=[jax.ShapeDtypeStruct((B, H), jnp.float32),
                   jax.ShapeDtypeStruct((B, H), jnp.float32)],
        compiler_params=pltpu.CompilerParams(
            dimension_semantics=("parallel",),
            allow_input_fusion=[True] * 13,
        ),
    )(input, hx, ts2, Wb, bb2, W_ff1, b1, W_ff2, b2, W_ta, ba, W_tb, btb)
    return (out[0], out[1])


# speedup vs baseline: 1.1349x; 1.1349x over previous
import jax
import jax.numpy as jnp
from jax.experimental import pallas as pl
from jax.experimental.pallas import tpu as pltpu

B, I, H, U = 4096, 128, 512, 512
TB = 1024


def _bf(a):
    return a.astype(jnp.bfloat16)


def _cfc_kernel(inp_ref, hx_ref, ts_ref, wb_ref, bb_ref,
                w1_ref, b1_ref, w2_ref, b2_ref,
                wa_ref, ba_ref, wtb_ref, btb_ref, out_ref, out2_ref):
    wb = _bf(wb_ref[...])
    w1 = _bf(w1_ref[...])
    w2 = _bf(w2_ref[...])
    wa = _bf(wa_ref[...])
    wtb = _bf(wtb_ref[...])
    bb = bb_ref[...][None, :]
    b1 = b1_ref[...][None, :]
    b2 = b2_ref[...][None, :]
    ba = ba_ref[...][None, :]
    btb = btb_ref[...][None, :]
    ts = ts_ref[...].reshape(TB, 1)
    half = TB // 2
    for k in range(2):
        rows = slice(k * half, (k + 1) * half)
        x = jnp.tanh(
            jnp.dot(_bf(inp_ref[rows, :]), wb[:I],
                    preferred_element_type=jnp.float32)
            + jnp.dot(_bf(hx_ref[rows, :]), wb[I:],
                      preferred_element_type=jnp.float32)
            + bb
        )
        xb = _bf(x)
        ff1 = jnp.tanh(jnp.dot(xb, w1,
                               preferred_element_type=jnp.float32) + b1)
        ff2 = jnp.tanh(jnp.dot(xb, w2,
                               preferred_element_type=jnp.float32) + b2)
        t_a = jnp.dot(xb, wa,
                      preferred_element_type=jnp.float32) + ba
        t_b = jnp.dot(xb, wtb,
                      preferred_element_type=jnp.float32) + btb
        s = jax.nn.sigmoid(t_a * ts[rows, :] + t_b)
        res = ff1 + s * (ff2 - ff1)
        out_ref[rows, :] = res
        out2_ref[rows, :] = res


def kernel(input, hx, ts, Wb, bb, W_ff1, b_ff1, W_ff2, b_ff2, W_ta, b_ta, W_tb, b_tb):
    whole = lambda shape: pl.BlockSpec(shape, lambda i: (0,) * len(shape))
    out = pl.pallas_call(
        _cfc_kernel,
        grid=(B // TB,),
        in_specs=[
            pl.BlockSpec((TB, I), lambda i: (i, 0)),
            pl.BlockSpec((TB, H), lambda i: (i, 0)),
            pl.BlockSpec((TB,), lambda i: (i,)),
            whole((I + H, U)),
            whole((U,)),
            whole((U, H)), whole((H,)),
            whole((U, H)), whole((H,)),
            whole((U, H)), whole((H,)),
            whole((U, H)), whole((H,)),
        ],
        out_specs=[pl.BlockSpec((TB, H), lambda i: (i, 0)),
                   pl.BlockSpec((TB, H), lambda i: (i, 0))],
        out_shape=[jax.ShapeDtypeStruct((B, H), jnp.float32),
                   jax.ShapeDtypeStruct((B, H), jnp.float32)],
        compiler_params=pltpu.CompilerParams(
            dimension_semantics=("parallel",),
        ),
    )(input, hx, ts, Wb, bb, W_ff1, b_ff1, W_ff2, b_ff2, W_ta, b_ta, W_tb, b_tb)
    return (out[0], out[1])
